# Initial kernel scaffold; baseline (speedup 1.0000x reference)
import jax
import jax.numpy as jnp
from jax.experimental import pallas as pl


def kernel(indices, values, factor_table, W_proj, b_proj):
    raise NotImplementedError("write your pallas kernel here")


# speedup vs baseline: 11.2682x; 11.2682x over previous
import functools

import jax
import jax.numpy as jnp
from jax.experimental import pallas as pl
from jax.experimental.pallas import tpu as pltpu

B = 16384
F = 12
D_MODEL = 768
FACTOR_DIM = 64
BLK = 2048


def _body(idx_ref, val_ref, ft_ref, w_ref, b_ref, out_ref):
    idx = idx_ref[...]
    v = jnp.clip(val_ref[...], 0.0, 1.0)
    kcol = jax.lax.broadcasted_iota(jnp.int32, (1, F), 1)
    wmat = jnp.zeros((idx.shape[0], F), jnp.float32)
    for f in range(F):
        onehot = idx[:, f:f + 1] == kcol
        wmat = wmat + jnp.where(onehot, v[:, f:f + 1], 0.0)
    m = jnp.dot(ft_ref[...], w_ref[...],
                preferred_element_type=jnp.float32)
    out_ref[...] = jnp.dot(wmat, m,
                           preferred_element_type=jnp.float32) + b_ref[...]


@jax.jit
def _run(indices, values, factor_table, W_proj, b_proj2d):
    grid = B // BLK
    return pl.pallas_call(
        _body,
        grid=(grid,),
        in_specs=[
            pl.BlockSpec((BLK, F), lambda i: (i, 0)),
            pl.BlockSpec((BLK, F), lambda i: (i, 0)),
            pl.BlockSpec((F, FACTOR_DIM), lambda i: (0, 0)),
            pl.BlockSpec((FACTOR_DIM, D_MODEL), lambda i: (0, 0)),
            pl.BlockSpec((1, D_MODEL), lambda i: (0, 0)),
        ],
        out_specs=pl.BlockSpec((BLK, D_MODEL), lambda i: (i, 0)),
        out_shape=jax.ShapeDtypeStruct((B, D_MODEL), jnp.float32),
    )(indices, values, factor_table, W_proj, b_proj2d)


def kernel(indices, values, factor_table, W_proj, b_proj):
    return _run(indices, values, factor_table, W_proj,
                b_proj.reshape(1, D_MODEL))
